# baseline (speedup 1.0000x reference)
import functools

import jax
import jax.numpy as jnp
from jax import lax
from jax.experimental import pallas as pl
from jax.experimental.pallas import tpu as pltpu
from jax.experimental.pallas import tpu_sc as plsc

NC = 2
NS = 16
NW = NC * NS
K = 128
NBUF = 2


def _acc_rows(n_nodes):
  return ((n_nodes + 1 + NS * 8 - 1) // (NS * 8)) * (NS * 8)


def _sc_segment_sum(n_nodes, d, e):
  assert e % K == 0
  chunks = e // K
  full = chunks // NW
  extra = chunks % NW
  np_rows = _acc_rows(n_nodes)
  zrows = np_rows // NS

  mesh = plsc.VectorSubcoreMesh(
      core_axis_name="c", subcore_axis_name="s", num_cores=NC,
      num_subcores=NS)

  scratch = [
      [pltpu.VMEM((K,), jnp.int32)] * NBUF,
      [pltpu.VMEM((K,), jnp.int32)] * NBUF,
      [pltpu.VMEM((K, d), jnp.float32)] * NBUF,
      [pltpu.SemaphoreType.DMA] * NBUF,
      [pltpu.SemaphoreType.DMA] * NBUF,
      pltpu.VMEM_SHARED((np_rows, d), jnp.float32),
  ]

  @functools.partial(
      pl.kernel,
      out_type=jax.ShapeDtypeStruct((NC, np_rows, d), jnp.float32),
      mesh=mesh,
      scratch_types=scratch,
  )
  def seg_sum(x_hbm, src_hbm, dst_hbm, z_hbm, out_hbm,
              sidx, didx, rows_bufs, gsem, isem, agg_sh):
    c = lax.axis_index("c")
    s = lax.axis_index("s")
    wid = c * NS + s

    pltpu.sync_copy(z_hbm, agg_sh.at[pl.ds(s * zrows, zrows)])
    plsc.subcore_barrier()

    def fetch(j, b):
      ofs = pl.ds((wid + j * NW) * K, K)
      pltpu.async_copy(src_hbm.at[ofs], sidx[b], isem[b])
      pltpu.async_copy(dst_hbm.at[ofs], didx[b], isem[b])

    def fetch_wait(j, b):
      ofs = pl.ds((wid + j * NW) * K, K)
      pltpu.make_async_copy(src_hbm.at[ofs], sidx[b], isem[b]).wait()
      pltpu.make_async_copy(dst_hbm.at[ofs], didx[b], isem[b]).wait()

    def gather(b):
      pltpu.async_copy(x_hbm.at[sidx[b]], rows_bufs[b], gsem[b])

    def gather_wait(b):
      pltpu.make_async_copy(x_hbm.at[sidx[b]], rows_bufs[b],
                            gsem[b]).wait()

    for b in range(NBUF):
      fetch(b, b)
    for b in range(NBUF):
      fetch_wait(b, b)
      gather(b)

    def body(jj, carry):
      for b in range(NBUF):
        j = jj * NBUF + b
        nxt = j + NBUF

        @pl.when(nxt < full)
        def _(nxt=nxt, b=b):
          fetch(nxt, b)

        gather_wait(b)
        pltpu.sync_copy(rows_bufs[b], agg_sh.at[didx[b]], add=True)

        @pl.when(nxt < full)
        def _(nxt=nxt, b=b):
          fetch_wait(nxt, b)
          gather(b)

      return carry

    assert full % NBUF == 0
    lax.fori_loop(0, full // NBUF, body, 0)

    if extra:
      @pl.when(wid < extra)
      def _():
        fetch(full, 0)
        fetch_wait(full, 0)
        gather(0)
        gather_wait(0)
        pltpu.sync_copy(rows_bufs[0], agg_sh.at[didx[0]], add=True)

    plsc.subcore_barrier()

    r0 = s * zrows
    pltpu.sync_copy(agg_sh.at[pl.ds(r0, zrows)],
                    out_hbm.at[c].at[pl.ds(r0, zrows)])

  return seg_sum


def _tc_finish_body(x_ref, agg_ref, u_ref, v_ref, o_ref):
  agg = agg_ref[0] + agg_ref[1]
  h = (jnp.dot(x_ref[...], u_ref[...], preferred_element_type=jnp.float32)
       + jnp.dot(agg, v_ref[...], preferred_element_type=jnp.float32))
  h = jnp.maximum(h, 0.0)
  m = jnp.max(h, axis=-1, keepdims=True)
  e = jnp.exp(h - m)
  o_ref[...] = e / jnp.sum(e, axis=-1, keepdims=True)


def kernel(x, edge_index, U, V):
  n, d = x.shape
  e = edge_index.shape[1]

  np_rows = _acc_rows(n)
  zeros = jnp.zeros((np_rows // NS, d), jnp.float32)

  agg2 = _sc_segment_sum(n, d, e)(x, edge_index[0], edge_index[1], zeros)

  blk = 1000
  grid = n // blk
  out = pl.pallas_call(
      _tc_finish_body,
      grid=(grid,),
      in_specs=[
          pl.BlockSpec((blk, d), lambda i: (i, 0)),
          pl.BlockSpec((NC, blk, d), lambda i: (0, i, 0)),
          pl.BlockSpec((d, d), lambda i: (0, 0)),
          pl.BlockSpec((d, d), lambda i: (0, 0)),
      ],
      out_specs=pl.BlockSpec((blk, d), lambda i: (i, 0)),
      out_shape=jax.ShapeDtypeStruct((n, d), jnp.float32),
  )(x, agg2, U, V)
  return out


# speedup vs baseline: 1.0001x; 1.0001x over previous
import functools

import jax
import jax.numpy as jnp
from jax import lax
from jax.experimental import pallas as pl
from jax.experimental.pallas import tpu as pltpu
from jax.experimental.pallas import tpu_sc as plsc

NC = 2
NS = 16
NW = NC * NS
K = 128
NBUF = 2


def _acc_rows(n_nodes):
  return ((n_nodes + 1 + NS * 8 - 1) // (NS * 8)) * (NS * 8)


def _sc_segment_sum(n_nodes, d, e):
  assert e % K == 0
  chunks = e // K
  full = chunks // NW
  extra = chunks % NW
  np_rows = _acc_rows(n_nodes)
  zrows = np_rows // NS

  mesh = plsc.VectorSubcoreMesh(
      core_axis_name="c", subcore_axis_name="s", num_cores=NC,
      num_subcores=NS)

  nidx = 2 * NBUF
  scratch = [
      pltpu.VMEM((nidx, K), jnp.int32),
      pltpu.VMEM((nidx, K), jnp.int32),
      [pltpu.VMEM((K, d), jnp.float32)] * NBUF,
      [pltpu.SemaphoreType.DMA] * NBUF,
      [pltpu.SemaphoreType.DMA] * nidx,
      pltpu.VMEM_SHARED((np_rows, d), jnp.float32),
  ]

  @functools.partial(
      pl.kernel,
      out_type=jax.ShapeDtypeStruct((NC, np_rows, d), jnp.float32),
      mesh=mesh,
      scratch_types=scratch,
  )
  def seg_sum(x_hbm, src_hbm, dst_hbm, z_hbm, out_hbm,
              sidx2, didx2, rows_bufs, gsem, isem, agg_sh):
    sidx = [sidx2.at[q] for q in range(nidx)]
    didx = [didx2.at[q] for q in range(nidx)]
    c = lax.axis_index("c")
    s = lax.axis_index("s")
    wid = c * NS + s

    pltpu.sync_copy(z_hbm, agg_sh.at[pl.ds(s * zrows, zrows)])
    plsc.subcore_barrier()

    def fetch(j, q):
      ofs = pl.ds((wid + j * NW) * K, K)
      pltpu.async_copy(src_hbm.at[ofs], sidx[q], isem[q])
      pltpu.async_copy(dst_hbm.at[ofs], didx[q], isem[q])

    def fetch_wait(j, q):
      ofs = pl.ds((wid + j * NW) * K, K)
      pltpu.make_async_copy(src_hbm.at[ofs], sidx[q], isem[q]).wait()
      pltpu.make_async_copy(dst_hbm.at[ofs], didx[q], isem[q]).wait()

    def gather(b, q):
      pltpu.async_copy(x_hbm.at[sidx[q]], rows_bufs[b], gsem[b])

    def gather_wait(b, q):
      pltpu.make_async_copy(x_hbm.at[sidx[q]], rows_bufs[b],
                            gsem[b]).wait()

    def consume(b, q):
      pltpu.sync_copy(rows_bufs[b], agg_sh.at[didx[q]], add=True)

    for b in range(NBUF):
      fetch(b, b)
    for b in range(NBUF):
      fetch_wait(b, b)
      gather(b, b)

    def body(jj, carry):
      for u in range(nidx):
        j = jj * nidx + u
        b = u % NBUF
        qn = (u + NBUF) % nidx
        fetch(j + NBUF, qn)
        gather_wait(b, u)
        consume(b, u)
        fetch_wait(j + NBUF, qn)
        gather(b, qn)
      return carry

    groups = (full - NBUF) // nidx
    lax.fori_loop(0, groups, body, 0)

    for j in range(groups * nidx, full):
      u = j % nidx
      b = j % NBUF
      gather_wait(b, u)
      consume(b, u)
      nxt = j + NBUF
      if nxt < full:
        fetch(nxt, nxt % nidx)
        fetch_wait(nxt, nxt % nidx)
        gather(b, nxt % nidx)
    if extra:
      @pl.when(wid < extra)
      def _():
        q = full % nidx
        fetch(full, q)
        fetch_wait(full, q)
        gather(0, q)
        gather_wait(0, q)
        consume(0, q)

    plsc.subcore_barrier()

    r0 = s * zrows
    pltpu.sync_copy(agg_sh.at[pl.ds(r0, zrows)],
                    out_hbm.at[c].at[pl.ds(r0, zrows)])

  return seg_sum


def _tc_finish_body(x_ref, agg_ref, u_ref, v_ref, o_ref):
  agg = agg_ref[0] + agg_ref[1]
  h = (jnp.dot(x_ref[...], u_ref[...], preferred_element_type=jnp.float32)
       + jnp.dot(agg, v_ref[...], preferred_element_type=jnp.float32))
  h = jnp.maximum(h, 0.0)
  m = jnp.max(h, axis=-1, keepdims=True)
  e = jnp.exp(h - m)
  o_ref[...] = e / jnp.sum(e, axis=-1, keepdims=True)


def kernel(x, edge_index, U, V):
  n, d = x.shape
  e = edge_index.shape[1]

  np_rows = _acc_rows(n)
  zeros = jnp.zeros((np_rows // NS, d), jnp.float32)

  agg2 = _sc_segment_sum(n, d, e)(x, edge_index[0], edge_index[1], zeros)

  blk = 1000
  grid = n // blk
  out = pl.pallas_call(
      _tc_finish_body,
      grid=(grid,),
      in_specs=[
          pl.BlockSpec((blk, d), lambda i: (i, 0)),
          pl.BlockSpec((NC, blk, d), lambda i: (0, i, 0)),
          pl.BlockSpec((d, d), lambda i: (0, 0)),
          pl.BlockSpec((d, d), lambda i: (0, 0)),
      ],
      out_specs=pl.BlockSpec((blk, d), lambda i: (i, 0)),
      out_shape=jax.ShapeDtypeStruct((n, d), jnp.float32),
  )(x, agg2, U, V)
  return out


# speedup vs baseline: 1.1111x; 1.1109x over previous
import functools

import jax
import jax.numpy as jnp
from jax import lax
from jax.experimental import pallas as pl
from jax.experimental.pallas import tpu as pltpu
from jax.experimental.pallas import tpu_sc as plsc

NC = 2
NS = 16
NW = NC * NS
K = 128
NBUF = 2


def _acc_rows(n_nodes):
  return ((n_nodes + 1 + NS * 8 - 1) // (NS * 8)) * (NS * 8)


def _sc_segment_sum(n_nodes, d, e):
  assert e % K == 0
  chunks = e // K
  full = chunks // NW
  extra = chunks % NW
  np_rows = _acc_rows(n_nodes)
  zrows = np_rows // NS

  mesh = plsc.VectorSubcoreMesh(
      core_axis_name="c", subcore_axis_name="s", num_cores=NC,
      num_subcores=NS)

  nidx = 2 * NBUF
  scratch = [
      pltpu.VMEM((nidx, 2, K), jnp.int32),
      [pltpu.VMEM((K, d), jnp.float32)] * NBUF,
      [pltpu.SemaphoreType.DMA] * NBUF,
      [pltpu.SemaphoreType.DMA] * nidx,
      pltpu.VMEM_SHARED((np_rows, d), jnp.float32),
  ]

  @functools.partial(
      pl.kernel,
      out_type=jax.ShapeDtypeStruct((NC, np_rows, d), jnp.float32),
      mesh=mesh,
      scratch_types=scratch,
  )
  def seg_sum(x_hbm, ei_hbm, z_hbm, out_hbm,
              idx2, rows_bufs, gsem, isem, agg_sh):
    sidx = [idx2.at[q].at[0] for q in range(nidx)]
    didx = [idx2.at[q].at[1] for q in range(nidx)]
    c = lax.axis_index("c")
    s = lax.axis_index("s")
    wid = c * NS + s

    pltpu.sync_copy(z_hbm, agg_sh.at[pl.ds(s * zrows, zrows)])
    plsc.subcore_barrier()

    def fetch(j, q):
      ofs = pl.ds((wid + j * NW) * K, K)
      pltpu.async_copy(ei_hbm.at[:, ofs], idx2.at[q], isem[q])

    def fetch_wait(j, q):
      ofs = pl.ds((wid + j * NW) * K, K)
      pltpu.make_async_copy(ei_hbm.at[:, ofs], idx2.at[q], isem[q]).wait()

    def gather(b, q):
      pltpu.async_copy(x_hbm.at[sidx[q]], rows_bufs[b], gsem[b])

    def gather_wait(b, q):
      pltpu.make_async_copy(x_hbm.at[sidx[q]], rows_bufs[b],
                            gsem[b]).wait()

    def consume(b, q):
      pltpu.sync_copy(rows_bufs[b], agg_sh.at[didx[q]], add=True)

    for b in range(NBUF):
      fetch(b, b)
    for b in range(NBUF):
      fetch_wait(b, b)
      gather(b, b)

    def body(jj, carry):
      for u in range(nidx):
        j = jj * nidx + u
        b = u % NBUF
        qn = (u + NBUF) % nidx
        fetch(j + NBUF, qn)
        gather_wait(b, u)
        consume(b, u)
        fetch_wait(j + NBUF, qn)
        gather(b, qn)
      return carry

    groups = (full - NBUF) // nidx
    lax.fori_loop(0, groups, body, 0)

    for j in range(groups * nidx, full):
      u = j % nidx
      b = j % NBUF
      gather_wait(b, u)
      consume(b, u)
      nxt = j + NBUF
      if nxt < full:
        fetch(nxt, nxt % nidx)
        fetch_wait(nxt, nxt % nidx)
        gather(b, nxt % nidx)
    if extra:
      @pl.when(wid < extra)
      def _():
        q = full % nidx
        fetch(full, q)
        fetch_wait(full, q)
        gather(0, q)
        gather_wait(0, q)
        consume(0, q)

    plsc.subcore_barrier()

    r0 = s * zrows
    pltpu.sync_copy(agg_sh.at[pl.ds(r0, zrows)],
                    out_hbm.at[c].at[pl.ds(r0, zrows)])

  return seg_sum


def _tc_finish_body(x_ref, agg_ref, u_ref, v_ref, o_ref):
  agg = agg_ref[0] + agg_ref[1]
  h = (jnp.dot(x_ref[...], u_ref[...], preferred_element_type=jnp.float32)
       + jnp.dot(agg, v_ref[...], preferred_element_type=jnp.float32))
  h = jnp.maximum(h, 0.0)
  m = jnp.max(h, axis=-1, keepdims=True)
  e = jnp.exp(h - m)
  o_ref[...] = e / jnp.sum(e, axis=-1, keepdims=True)


def kernel(x, edge_index, U, V):
  n, d = x.shape
  e = edge_index.shape[1]

  np_rows = _acc_rows(n)
  zeros = jnp.zeros((np_rows // NS, d), jnp.float32)

  agg2 = _sc_segment_sum(n, d, e)(x, edge_index, zeros)

  blk = 2000
  grid = n // blk
  out = pl.pallas_call(
      _tc_finish_body,
      grid=(grid,),
      in_specs=[
          pl.BlockSpec((blk, d), lambda i: (i, 0)),
          pl.BlockSpec((NC, blk, d), lambda i: (0, i, 0)),
          pl.BlockSpec((d, d), lambda i: (0, 0)),
          pl.BlockSpec((d, d), lambda i: (0, 0)),
      ],
      out_specs=pl.BlockSpec((blk, d), lambda i: (i, 0)),
      out_shape=jax.ShapeDtypeStruct((n, d), jnp.float32),
  )(x, agg2, U, V)
  return out


# speedup vs baseline: 1.2105x; 1.0895x over previous
import functools

import jax
import jax.numpy as jnp
from jax import lax
from jax.experimental import pallas as pl
from jax.experimental.pallas import tpu as pltpu
from jax.experimental.pallas import tpu_sc as plsc

NC = 2
NS = 16
NW = NC * NS
K = 128
NBUF = 3


def _acc_rows(n_nodes):
  return ((n_nodes + 7) // 8) * 8


def _sc_segment_sum(n_nodes, d, e):
  assert e % K == 0
  chunks = e // K
  full = chunks // NW
  extra = chunks % NW
  np_rows = _acc_rows(n_nodes)
  zbase = np_rows // (NS * 8) * 8
  zrem = (np_rows - zbase * NS) // 8
  zbig = zbase + 8

  mesh = plsc.VectorSubcoreMesh(
      core_axis_name="c", subcore_axis_name="s", num_cores=NC,
      num_subcores=NS)

  nidx = NBUF + 1
  group = NBUF * nidx
  scratch = [
      pltpu.VMEM((nidx, 2, K), jnp.int32),
      [pltpu.VMEM((K, d), jnp.float32)] * NBUF,
      [pltpu.SemaphoreType.DMA] * NBUF,
      [pltpu.SemaphoreType.DMA] * nidx,
      pltpu.VMEM_SHARED((np_rows, d), jnp.float32),
  ]

  @functools.partial(
      pl.kernel,
      out_type=jax.ShapeDtypeStruct((NC, np_rows, d), jnp.float32),
      mesh=mesh,
      scratch_types=scratch,
  )
  def seg_sum(x_hbm, ei_hbm, z_hbm, out_hbm,
              idx2, rows_bufs, gsem, isem, agg_sh):
    sidx = [idx2.at[q].at[0] for q in range(nidx)]
    didx = [idx2.at[q].at[1] for q in range(nidx)]
    c = lax.axis_index("c")
    s = lax.axis_index("s")
    wid = c * NS + s
    r0 = pl.multiple_of(s * zbase + jnp.minimum(s, zrem) * 8, 8)

    def fetch(j, q):
      ofs = pl.ds((wid + j * NW) * K, K)
      pltpu.async_copy(ei_hbm.at[:, ofs], idx2.at[q], isem[q])

    def fetch_wait(j, q):
      ofs = pl.ds((wid + j * NW) * K, K)
      pltpu.make_async_copy(ei_hbm.at[:, ofs], idx2.at[q], isem[q]).wait()

    def gather(b, q):
      pltpu.async_copy(x_hbm.at[sidx[q]], rows_bufs[b], gsem[b])

    def gather_wait(b, q):
      pltpu.make_async_copy(x_hbm.at[sidx[q]], rows_bufs[b],
                            gsem[b]).wait()

    def consume(b, q):
      pltpu.sync_copy(rows_bufs[b], agg_sh.at[didx[q]], add=True)

    for b in range(NBUF):
      fetch(b, b)
    for b in range(NBUF):
      fetch_wait(b, b)
      gather(b, b)

    @pl.when(s < zrem)
    def _():
      pltpu.sync_copy(z_hbm, agg_sh.at[pl.ds(r0, zbig)])

    @pl.when(s >= zrem)
    def _():
      pltpu.sync_copy(z_hbm.at[pl.ds(0, zbase)],
                      agg_sh.at[pl.ds(r0, zbase)])

    plsc.subcore_barrier()

    def body(jj, carry):
      for u in range(group):
        j = jj * group + u
        b = u % NBUF
        q = u % nidx
        qn = (u + NBUF) % nidx
        fetch(j + NBUF, qn)
        gather_wait(b, q)
        consume(b, q)
        fetch_wait(j + NBUF, qn)
        gather(b, qn)
      return carry

    groups = (full - NBUF) // group
    lax.fori_loop(0, groups, body, 0)

    for j in range(groups * group, full):
      u = j % nidx
      b = j % NBUF
      gather_wait(b, u)
      consume(b, u)
      nxt = j + NBUF
      if nxt < full:
        fetch(nxt, nxt % nidx)
        fetch_wait(nxt, nxt % nidx)
        gather(b, nxt % nidx)
    if extra:
      @pl.when(wid < extra)
      def _():
        q = full % nidx
        fetch(full, q)
        fetch_wait(full, q)
        gather(0, q)
        gather_wait(0, q)
        consume(0, q)

    plsc.subcore_barrier()

    @pl.when(s < zrem)
    def _():
      pltpu.sync_copy(agg_sh.at[pl.ds(r0, zbig)],
                      out_hbm.at[c].at[pl.ds(r0, zbig)])

    @pl.when(s >= zrem)
    def _():
      pltpu.sync_copy(agg_sh.at[pl.ds(r0, zbase)],
                      out_hbm.at[c].at[pl.ds(r0, zbase)])

  return seg_sum


def _tc_finish_body(x_ref, agg_ref, u_ref, v_ref, o_ref):
  agg = agg_ref[0] + agg_ref[1]
  h = (jnp.dot(x_ref[...], u_ref[...], preferred_element_type=jnp.float32)
       + jnp.dot(agg, v_ref[...], preferred_element_type=jnp.float32))
  h = jnp.maximum(h, 0.0)
  m = jnp.max(h, axis=-1, keepdims=True)
  e = jnp.exp(h - m)
  o_ref[...] = e / jnp.sum(e, axis=-1, keepdims=True)


def kernel(x, edge_index, U, V):
  n, d = x.shape
  e = edge_index.shape[1]

  np_rows = _acc_rows(n)
  zbig = np_rows // (NS * 8) * 8 + 8
  zeros = jnp.zeros((zbig, d), jnp.float32)

  agg2 = _sc_segment_sum(n, d, e)(x, edge_index, zeros)

  blk = 2000
  grid = n // blk
  out = pl.pallas_call(
      _tc_finish_body,
      grid=(grid,),
      in_specs=[
          pl.BlockSpec((blk, d), lambda i: (i, 0)),
          pl.BlockSpec((NC, blk, d), lambda i: (0, i, 0)),
          pl.BlockSpec((d, d), lambda i: (0, 0)),
          pl.BlockSpec((d, d), lambda i: (0, 0)),
      ],
      out_specs=pl.BlockSpec((blk, d), lambda i: (i, 0)),
      out_shape=jax.ShapeDtypeStruct((n, d), jnp.float32),
  )(x, agg2, U, V)
  return out


# speedup vs baseline: 1.2243x; 1.0114x over previous
import functools

import jax
import jax.numpy as jnp
from jax import lax
from jax.experimental import pallas as pl
from jax.experimental.pallas import tpu as pltpu
from jax.experimental.pallas import tpu_sc as plsc

NC = 2
NS = 16
NW = NC * NS
K = 128
NBUF = 3


def _acc_rows(n_nodes):
  return ((n_nodes + 7) // 8) * 8


def _sc_segment_sum(n_nodes, d, e):
  assert e % K == 0
  chunks = e // K
  full = chunks // NW
  extra = chunks % NW
  np_rows = _acc_rows(n_nodes)
  zbase = np_rows // (NS * 8) * 8
  zrem = (np_rows - zbase * NS) // 8
  zbig = zbase + 8

  mesh = plsc.VectorSubcoreMesh(
      core_axis_name="c", subcore_axis_name="s", num_cores=NC,
      num_subcores=NS)

  nidx = NBUF + 1
  group = NBUF * nidx
  scratch = [
      pltpu.VMEM((nidx, 2, K), jnp.int32),
      [pltpu.VMEM((K, d), jnp.float32)] * NBUF,
      [pltpu.SemaphoreType.DMA] * NBUF,
      [pltpu.SemaphoreType.DMA] * nidx,
      pltpu.VMEM_SHARED((np_rows, d), jnp.float32),
  ]

  @functools.partial(
      pl.kernel,
      out_type=jax.ShapeDtypeStruct((NC, np_rows, d), jnp.float32),
      mesh=mesh,
      scratch_types=scratch,
  )
  def seg_sum(x_hbm, ei_hbm, z_hbm, out_hbm,
              idx2, rows_bufs, gsem, isem, agg_sh):
    sidx = [idx2.at[q].at[0] for q in range(nidx)]
    didx = [idx2.at[q].at[1] for q in range(nidx)]
    c = lax.axis_index("c")
    s = lax.axis_index("s")
    wid = c * NS + s
    r0 = pl.multiple_of(s * zbase + jnp.minimum(s, zrem) * 8, 8)

    def fetch(j, q):
      ofs = pl.ds((wid + j * NW) * K, K)
      pltpu.async_copy(ei_hbm.at[:, ofs], idx2.at[q], isem[q])

    def fetch_wait(j, q):
      ofs = pl.ds((wid + j * NW) * K, K)
      pltpu.make_async_copy(ei_hbm.at[:, ofs], idx2.at[q], isem[q]).wait()

    def gather(b, q):
      pltpu.async_copy(x_hbm.at[sidx[q]], rows_bufs[b], gsem[b])

    def gather_wait(b, q):
      pltpu.make_async_copy(x_hbm.at[sidx[q]], rows_bufs[b],
                            gsem[b]).wait()

    def consume(b, q):
      pltpu.sync_copy(rows_bufs[b], agg_sh.at[didx[q]], add=True)

    for b in range(NBUF):
      fetch(b, b)
    for b in range(NBUF):
      fetch_wait(b, b)
      gather(b, b)

    @pl.when(s < zrem)
    def _():
      pltpu.sync_copy(z_hbm, agg_sh.at[pl.ds(r0, zbig)])

    @pl.when(s >= zrem)
    def _():
      pltpu.sync_copy(z_hbm.at[pl.ds(0, zbase)],
                      agg_sh.at[pl.ds(r0, zbase)])

    plsc.subcore_barrier()

    def body(jj, carry):
      for u in range(group):
        j = jj * group + u
        b = u % NBUF
        q = u % nidx
        qn = (u + NBUF) % nidx
        fetch(j + NBUF, qn)
        gather_wait(b, q)
        consume(b, q)
        fetch_wait(j + NBUF, qn)
        gather(b, qn)
      return carry

    groups = (full - NBUF) // group
    lax.fori_loop(0, groups, body, 0)

    qx = full % nidx
    for j in range(groups * group, full):
      u = j % nidx
      b = j % NBUF
      gather_wait(b, u)
      consume(b, u)
      nxt = j + NBUF
      if nxt < full:
        fetch(nxt, nxt % nidx)
        fetch_wait(nxt, nxt % nidx)
        gather(b, nxt % nidx)
      if extra and j == full - NBUF:
        @pl.when(wid < extra)
        def _():
          fetch(full, qx)
      if extra and j == full - NBUF + 1:
        @pl.when(wid < extra)
        def _():
          fetch_wait(full, qx)
          gather(full % NBUF, qx)
    if extra:
      @pl.when(wid < extra)
      def _():
        gather_wait(full % NBUF, qx)
        consume(full % NBUF, qx)

    plsc.subcore_barrier()

    @pl.when(s < zrem)
    def _():
      pltpu.sync_copy(agg_sh.at[pl.ds(r0, zbig)],
                      out_hbm.at[c].at[pl.ds(r0, zbig)])

    @pl.when(s >= zrem)
    def _():
      pltpu.sync_copy(agg_sh.at[pl.ds(r0, zbase)],
                      out_hbm.at[c].at[pl.ds(r0, zbase)])

  return seg_sum


def _tc_finish_body(x_ref, agg_ref, u_ref, v_ref, o_ref):
  agg = agg_ref[0] + agg_ref[1]
  h = (jnp.dot(x_ref[...], u_ref[...], preferred_element_type=jnp.float32)
       + jnp.dot(agg, v_ref[...], preferred_element_type=jnp.float32))
  h = jnp.maximum(h, 0.0)
  m = jnp.max(h, axis=-1, keepdims=True)
  e = jnp.exp(h - m)
  o_ref[...] = e / jnp.sum(e, axis=-1, keepdims=True)


def kernel(x, edge_index, U, V):
  n, d = x.shape
  e = edge_index.shape[1]

  np_rows = _acc_rows(n)
  zbig = np_rows // (NS * 8) * 8 + 8
  zeros = jnp.zeros((zbig, d), jnp.float32)

  agg2 = _sc_segment_sum(n, d, e)(x, edge_index, zeros)

  blk = 2000
  grid = n // blk
  out = pl.pallas_call(
      _tc_finish_body,
      grid=(grid,),
      in_specs=[
          pl.BlockSpec((blk, d), lambda i: (i, 0)),
          pl.BlockSpec((NC, blk, d), lambda i: (0, i, 0)),
          pl.BlockSpec((d, d), lambda i: (0, 0)),
          pl.BlockSpec((d, d), lambda i: (0, 0)),
      ],
      out_specs=pl.BlockSpec((blk, d), lambda i: (i, 0)),
      out_shape=jax.ShapeDtypeStruct((n, d), jnp.float32),
  )(x, agg2, U, V)
  return out
